# baseline (speedup 1.0000x reference)
import functools

import jax
import jax.numpy as jnp
from jax import lax
from jax.experimental import pallas as pl
from jax.experimental.pallas import tpu as pltpu
from jax.experimental.pallas import tpu_sc as plsc

_NUM_PASSES = 3
_NC = 2
_NS = 16
_LANES = 16


_DEPTH = 4


def _pick_chunk(epw: int, n: int) -> int:
    budget = ((131071 - 2 * n) // _DEPTH) * 2 // 3
    for c in range(budget - budget % 32, 31, -32):
        if epw % c == 0:
            return c
    return 2 * _LANES


def _make_propagate(B: int, N: int, E: int):
    NW = _NC * _NS
    R = NW // B
    EPW = E // R
    C = _pick_chunk(EPW, N)
    n_chunks = EPW // C
    D = _DEPTH
    full = n_chunks // D
    rem = n_chunks % D

    mesh = plsc.VectorSubcoreMesh(
        core_axis_name="c", subcore_axis_name="s",
        num_cores=_NC, num_subcores=_NS)

    @functools.partial(
        pl.kernel,
        out_type=jax.ShapeDtypeStruct((NW, N), jnp.float32),
        mesh=mesh,
        scratch_types=(
            [pltpu.VMEM((N,), jnp.float32),
             pltpu.VMEM((N,), jnp.float32)]
            + [pltpu.VMEM((3 * C // 2,), jnp.int32)] * D
            + [pltpu.SemaphoreType.DMA] * D
        ),
        compiler_params=pltpu.CompilerParams(needs_layout_passes=False),
    )
    def prop(h_hbm, pw_hbm, out_hbm, h_v, aggr_v, *ring):
        bufs = ring[:D]
        sems = ring[D:]
        cid = lax.axis_index("c")
        sid = lax.axis_index("s")
        wid = sid * _NC + cid
        b = wid % B
        base = (wid // B) * EPW

        def issue(chunk_idx, slot):
            off = pl.multiple_of((base + chunk_idx * C) * 3 // 2, 8)
            pltpu.async_copy(pw_hbm.at[pl.ds(off, 3 * C // 2)], bufs[slot],
                             sems[slot])

        for s in range(D - 1):
            issue(s, s)
        pltpu.sync_copy(h_hbm.at[b], h_v)

        def zero_body(i, carry):
            aggr_v[pl.ds(i * _LANES, _LANES)] = jnp.zeros((_LANES,),
                                                          jnp.float32)
            return carry
        lax.fori_loop(0, N // _LANES, zero_body, 0, unroll=8)

        def process(slot):
            buf = bufs[slot]
            pltpu.make_async_copy(pw_hbm.at[pl.ds(0, 3 * C // 2)], buf,
                                  sems[slot]).wait()

            @plsc.parallel_loop(0, C // 32, 1, unroll=4)
            def _(i):
                off = i * 32
                wp = plsc.bitcast(buf[pl.ds(C + i * _LANES, _LANES)],
                                  jnp.bfloat16)
                w0, w1 = plsc.unpack(wp, format=plsc.PackFormat.INTERLEAVED)
                for half, wi in ((0, w0), (1, w1)):
                    pr = buf[pl.ds(off + half * _LANES, _LANES)]
                    si = pr & 0xFFFF
                    di = lax.shift_right_logical(pr, 16)
                    vals = plsc.load_gather(h_v, [si])
                    plsc.addupdate_scatter(aggr_v, [di], vals * wi)

        def ring_body(m, carry):
            c0 = m * D
            for ph in range(D):
                nxt = c0 + ph + (D - 1)

                @pl.when(nxt < n_chunks)
                def _():
                    issue(nxt, (ph + D - 1) % D)
                process(ph)
            return carry
        lax.fori_loop(0, full, ring_body, 0)
        for ph in range(rem):
            process(ph)

        pltpu.sync_copy(aggr_v, out_hbm.at[wid])

    return prop


def _prep_edges(src, dst, edge_weight, edge_weight_multiplier, C):
    E = edge_weight.shape[0]
    G = E // C
    BG = 1
    for cand in (20, 10, 8, 5, 4, 2):
        if G % cand == 0:
            BG = cand
            break

    ew_p = edge_weight.reshape(-1, 2, _LANES).swapaxes(1, 2).reshape(E)
    mult_p = edge_weight_multiplier.reshape(-1, 2, _LANES).swapaxes(1, 2)
    mult_p = mult_p.reshape(E)

    def body(src_ref, dst_ref, ew_ref, mult_ref, pair_ref, wbf_ref):
        pair_ref[...] = (dst_ref[...] << 16) | src_ref[...]
        w = ew_ref[...] * jax.nn.sigmoid(mult_ref[...])
        wbf_ref[...] = w.astype(jnp.bfloat16)

    spec_i = pl.BlockSpec((BG, 1, C), lambda i: (i, 0, 0))
    pair, wbf = pl.pallas_call(
        body,
        grid=(G // BG,),
        in_specs=[spec_i] * 4,
        out_specs=[spec_i] * 2,
        out_shape=[jax.ShapeDtypeStruct((G, 1, C), jnp.int32),
                   jax.ShapeDtypeStruct((G, 1, C), jnp.bfloat16)],
    )(src.reshape(G, 1, C), dst.reshape(G, 1, C),
      ew_p.reshape(G, 1, C), mult_p.reshape(G, 1, C))
    wbits = lax.bitcast_convert_type(wbf.reshape(G, C // 2, 2), jnp.int32)
    packed = jnp.concatenate([pair.reshape(G, C), wbits], axis=1)
    return packed.reshape(3 * E // 2)


def _reduce_norm(parts, B):
    NW = parts.shape[0]
    R = NW // B
    aggr = parts[0:B]
    for k in range(1, R):
        aggr = aggr + parts[k * B:(k + 1) * B]
    mn = jnp.min(aggr)
    mx = jnp.max(aggr)
    return (aggr - mn) / (mx - mn)


def _make_update(B, N, NW):
    def body(parts_ref, thr_ref, h_ref):
        t = _reduce_norm(parts_ref[...], B)
        h_ref[...] = jax.nn.sigmoid(t - jnp.abs(thr_ref[...]))

    return pl.pallas_call(
        body,
        out_shape=jax.ShapeDtypeStruct((B, N), jnp.float32),
    )


def _make_final(B, N, NW, n_classes):
    def body(parts_ref, thr_ref, mask_ref, wfc_ref, bfc_ref, out_ref):
        t = _reduce_norm(parts_ref[...], B)
        h = jax.nn.sigmoid(t - jnp.abs(thr_ref[...]))
        pooled = jnp.sum(h * mask_ref[...], axis=1, keepdims=True)
        out_ref[...] = pooled * wfc_ref[...] + bfc_ref[...]

    return pl.pallas_call(
        body,
        out_shape=jax.ShapeDtypeStruct((B, n_classes), jnp.float32),
    )


def kernel(x, edge_index, edge_weight, edge_weight_multiplier,
           neuron_activation_threshold, W_fc, b_fc, sel_idx):
    N = neuron_activation_threshold.shape[0]
    B = x.shape[0] // N
    E = edge_weight.shape[0]
    S = sel_idx.shape[0]
    n_classes = W_fc.shape[0]
    NW = _NC * _NS

    h = x.reshape(B, N)
    src = edge_index[0]
    dst = edge_index[1]
    thr2 = neuron_activation_threshold.reshape(1, N)
    maskw = jnp.zeros((N,), jnp.float32).at[sel_idx].set(1.0 / S).reshape(1, N)
    wfc_row = W_fc.reshape(1, n_classes)
    bfc_row = b_fc.reshape(1, n_classes)

    C = _pick_chunk(E // (NW // B), N)
    pairw = _prep_edges(src, dst, edge_weight, edge_weight_multiplier, C)
    prop = _make_propagate(B, N, E)
    update = _make_update(B, N, NW)
    final = _make_final(B, N, NW, n_classes)

    for p in range(_NUM_PASSES):
        parts = prop(h, pairw)
        if p < _NUM_PASSES - 1:
            h = update(parts, thr2)
        else:
            out = final(parts, thr2, maskw, wfc_row, bfc_row)
    return out


# speedup vs baseline: 5.2689x; 5.2689x over previous
import functools

import jax
import jax.numpy as jnp
from jax import lax
from jax.experimental import pallas as pl
from jax.experimental.pallas import tpu as pltpu
from jax.experimental.pallas import tpu_sc as plsc

_NUM_PASSES = 3
_NC = 2
_NS = 16
_LANES = 16


_DEPTH = 4


def _pick_chunk(epw: int, n: int) -> int:
    budget = (131071 - 2 * n) // (2 * _DEPTH)
    for c in range(budget - budget % 16, 15, -16):
        if epw % c == 0:
            return c
    return _LANES


def _make_propagate(B: int, N: int, E: int):
    NW = _NC * _NS
    R = NW // B
    EPW = E // R
    C = _pick_chunk(EPW, N)
    n_chunks = EPW // C
    D = _DEPTH
    full = n_chunks // D
    rem = n_chunks % D

    mesh = plsc.VectorSubcoreMesh(
        core_axis_name="c", subcore_axis_name="s",
        num_cores=_NC, num_subcores=_NS)

    @functools.partial(
        pl.kernel,
        out_type=jax.ShapeDtypeStruct((NW, N), jnp.float32),
        mesh=mesh,
        scratch_types=(
            [pltpu.VMEM((N,), jnp.float32),
             pltpu.VMEM((N,), jnp.float32)]
            + [pltpu.VMEM((2 * C,), jnp.int32)] * D
            + [pltpu.SemaphoreType.DMA] * D
        ),
        compiler_params=pltpu.CompilerParams(needs_layout_passes=False),
    )
    def prop(h_hbm, pw_hbm, out_hbm, h_v, aggr_v, *ring):
        bufs = ring[:D]
        sems = ring[D:]
        cid = lax.axis_index("c")
        sid = lax.axis_index("s")
        wid = sid * _NC + cid
        b = wid % B
        base = (wid // B) * EPW

        def issue(chunk_idx, slot):
            off = (base + chunk_idx * C) * 2
            pltpu.async_copy(pw_hbm.at[pl.ds(off, 2 * C)], bufs[slot],
                             sems[slot])

        for s in range(D - 1):
            issue(s, s)
        pltpu.sync_copy(h_hbm.at[b], h_v)

        def zero_body(i, carry):
            aggr_v[pl.ds(i * _LANES, _LANES)] = jnp.zeros((_LANES,),
                                                          jnp.float32)
            return carry
        lax.fori_loop(0, N // _LANES, zero_body, 0, unroll=8)

        def process(slot):
            buf = bufs[slot]
            pltpu.make_async_copy(pw_hbm.at[pl.ds(0, 2 * C)], buf,
                                  sems[slot]).wait()

            @plsc.parallel_loop(0, C, _LANES, unroll=16)
            def _(off):
                pr = buf[pl.ds(off, _LANES)]
                wi = plsc.bitcast(buf[pl.ds(C + off, _LANES)], jnp.float32)
                si = pr & 0xFFFF
                di = lax.shift_right_logical(pr, 16)
                vals = plsc.load_gather(h_v, [si])
                plsc.addupdate_scatter(aggr_v, [di], vals * wi)

        def ring_body(m, carry):
            c0 = m * D
            for ph in range(D):
                nxt = c0 + ph + (D - 1)

                @pl.when(nxt < n_chunks)
                def _():
                    issue(nxt, (ph + D - 1) % D)
                process(ph)
            return carry
        lax.fori_loop(0, full, ring_body, 0)
        for ph in range(rem):
            process(ph)

        pltpu.sync_copy(aggr_v, out_hbm.at[wid])

    return prop


def _prep_edges(src, dst, edge_weight, edge_weight_multiplier, C):
    E = edge_weight.shape[0]
    G = E // C
    BG = 1
    for cand in (20, 10, 8, 5, 4, 2):
        if G % cand == 0:
            BG = cand
            break

    def body(src_ref, dst_ref, ew_ref, mult_ref, out_ref):
        pair = (dst_ref[...] << 16) | src_ref[...]
        w = ew_ref[...] * jax.nn.sigmoid(mult_ref[...])
        out_ref[:, 0:1, :] = pair
        out_ref[:, 1:2, :] = lax.bitcast_convert_type(w, jnp.int32)

    spec_i = pl.BlockSpec((BG, 1, C), lambda i: (i, 0, 0))
    out = pl.pallas_call(
        body,
        grid=(G // BG,),
        in_specs=[spec_i] * 4,
        out_specs=pl.BlockSpec((BG, 2, C), lambda i: (i, 0, 0)),
        out_shape=jax.ShapeDtypeStruct((G, 2, C), jnp.int32),
    )(src.reshape(G, 1, C), dst.reshape(G, 1, C),
      edge_weight.reshape(G, 1, C),
      edge_weight_multiplier.reshape(G, 1, C))
    return out.reshape(2 * E)


def _reduce_norm(parts, B):
    NW = parts.shape[0]
    R = NW // B
    aggr = parts[0:B]
    for k in range(1, R):
        aggr = aggr + parts[k * B:(k + 1) * B]
    mn = jnp.min(aggr)
    mx = jnp.max(aggr)
    return (aggr - mn) / (mx - mn)


def _make_update(B, N, NW):
    def body(parts_ref, thr_ref, h_ref):
        t = _reduce_norm(parts_ref[...], B)
        h_ref[...] = jax.nn.sigmoid(t - jnp.abs(thr_ref[...]))

    return pl.pallas_call(
        body,
        out_shape=jax.ShapeDtypeStruct((B, N), jnp.float32),
    )


def _make_final(B, N, NW, n_classes):
    def body(parts_ref, thr_ref, mask_ref, wfc_ref, bfc_ref, out_ref):
        t = _reduce_norm(parts_ref[...], B)
        h = jax.nn.sigmoid(t - jnp.abs(thr_ref[...]))
        pooled = jnp.sum(h * mask_ref[...], axis=1, keepdims=True)
        out_ref[...] = pooled * wfc_ref[...] + bfc_ref[...]

    return pl.pallas_call(
        body,
        out_shape=jax.ShapeDtypeStruct((B, n_classes), jnp.float32),
    )


def kernel(x, edge_index, edge_weight, edge_weight_multiplier,
           neuron_activation_threshold, W_fc, b_fc, sel_idx):
    N = neuron_activation_threshold.shape[0]
    B = x.shape[0] // N
    E = edge_weight.shape[0]
    S = sel_idx.shape[0]
    n_classes = W_fc.shape[0]
    NW = _NC * _NS

    h = x.reshape(B, N)
    src = edge_index[0]
    dst = edge_index[1]
    thr2 = neuron_activation_threshold.reshape(1, N)
    maskw = jnp.zeros((N,), jnp.float32).at[sel_idx].set(1.0 / S).reshape(1, N)
    wfc_row = W_fc.reshape(1, n_classes)
    bfc_row = b_fc.reshape(1, n_classes)

    C = _pick_chunk(E // (NW // B), N)
    pairw = _prep_edges(src, dst, edge_weight, edge_weight_multiplier, C)
    prop = _make_propagate(B, N, E)
    update = _make_update(B, N, NW)
    final = _make_final(B, N, NW, n_classes)

    for p in range(_NUM_PASSES):
        parts = prop(h, pairw)
        if p < _NUM_PASSES - 1:
            h = update(parts, thr2)
        else:
            out = final(parts, thr2, maskw, wfc_row, bfc_row)
    return out


# speedup vs baseline: 5.5788x; 1.0588x over previous
import functools

import jax
import jax.numpy as jnp
from jax import lax
from jax.experimental import pallas as pl
from jax.experimental.pallas import tpu as pltpu
from jax.experimental.pallas import tpu_sc as plsc

_NUM_PASSES = 3
_NC = 2
_NS = 16
_LANES = 16


_DEPTH = 4


def _pick_chunk(epw: int, n: int) -> int:
    budget = (131071 - 2 * n) // (2 * _DEPTH)
    for c in range(budget - budget % 16, 15, -16):
        if epw % c == 0:
            return c
    return _LANES


def _make_propagate(B: int, N: int, E: int):
    NW = _NC * _NS
    R = NW // B
    EPW = E // R
    C = _pick_chunk(EPW, N)
    n_chunks = EPW // C
    D = _DEPTH
    full = n_chunks // D
    rem = n_chunks % D

    mesh = plsc.VectorSubcoreMesh(
        core_axis_name="c", subcore_axis_name="s",
        num_cores=_NC, num_subcores=_NS)

    @functools.partial(
        pl.kernel,
        out_type=jax.ShapeDtypeStruct((NW, N), jnp.float32),
        mesh=mesh,
        scratch_types=(
            [pltpu.VMEM((N,), jnp.float32),
             pltpu.VMEM((N,), jnp.float32)]
            + [pltpu.VMEM((2 * C,), jnp.int32)] * D
            + [pltpu.SemaphoreType.DMA] * D
        ),
        compiler_params=pltpu.CompilerParams(needs_layout_passes=False),
    )
    def prop(h_hbm, pw_hbm, out_hbm, h_v, aggr_v, *ring):
        bufs = ring[:D]
        sems = ring[D:]
        cid = lax.axis_index("c")
        sid = lax.axis_index("s")
        wid = sid * _NC + cid
        b = wid % B
        base = (wid // B) * EPW

        def issue(chunk_idx, slot):
            off = (base + chunk_idx * C) * 2
            pltpu.async_copy(pw_hbm.at[pl.ds(off, 2 * C)], bufs[slot],
                             sems[slot])

        for s in range(D - 1):
            issue(s, s)
        pltpu.sync_copy(h_hbm.at[b], h_v)

        def zero_body(i, carry):
            aggr_v[pl.ds(i * _LANES, _LANES)] = jnp.zeros((_LANES,),
                                                          jnp.float32)
            return carry
        lax.fori_loop(0, N // _LANES, zero_body, 0, unroll=8)

        def process(slot):
            buf = bufs[slot]
            pltpu.make_async_copy(pw_hbm.at[pl.ds(0, 2 * C)], buf,
                                  sems[slot]).wait()

            @plsc.parallel_loop(0, C, _LANES, unroll=8)
            def _(off):
                pr = buf[pl.ds(off, _LANES)]
                wi = plsc.bitcast(buf[pl.ds(C + off, _LANES)], jnp.float32)
                si = pr & 0xFFFF
                di = lax.shift_right_logical(pr, 16)
                vals = plsc.load_gather(h_v, [si])
                plsc.addupdate_scatter(aggr_v, [di], vals * wi)

        def ring_body(m, carry):
            c0 = m * D
            for ph in range(D):
                nxt = c0 + ph + (D - 1)

                @pl.when(nxt < n_chunks)
                def _():
                    issue(nxt, (ph + D - 1) % D)
                process(ph)
            return carry
        lax.fori_loop(0, full, ring_body, 0)
        for ph in range(rem):
            process(ph)

        pltpu.sync_copy(aggr_v, out_hbm.at[wid])

    return prop


def _prep_edges(src, dst, edge_weight, edge_weight_multiplier, C):
    E = edge_weight.shape[0]
    G = E // C
    BG = 1
    for cand in (20, 10, 8, 5, 4, 2):
        if G % cand == 0:
            BG = cand
            break

    def body(src_ref, dst_ref, ew_ref, mult_ref, out_ref):
        pair = (dst_ref[...] << 16) | src_ref[...]
        w = ew_ref[...] * jax.nn.sigmoid(mult_ref[...])
        out_ref[:, 0:1, :] = pair
        out_ref[:, 1:2, :] = lax.bitcast_convert_type(w, jnp.int32)

    spec_i = pl.BlockSpec((BG, 1, C), lambda i: (i, 0, 0))
    out = pl.pallas_call(
        body,
        grid=(G // BG,),
        in_specs=[spec_i] * 4,
        out_specs=pl.BlockSpec((BG, 2, C), lambda i: (i, 0, 0)),
        out_shape=jax.ShapeDtypeStruct((G, 2, C), jnp.int32),
    )(src.reshape(G, 1, C), dst.reshape(G, 1, C),
      edge_weight.reshape(G, 1, C),
      edge_weight_multiplier.reshape(G, 1, C))
    return out.reshape(2 * E)


def _reduce_norm(parts, B):
    NW = parts.shape[0]
    R = NW // B
    aggr = parts[0:B]
    for k in range(1, R):
        aggr = aggr + parts[k * B:(k + 1) * B]
    mn = jnp.min(aggr)
    mx = jnp.max(aggr)
    return (aggr - mn) / (mx - mn)


def _make_update(B, N, NW):
    def body(parts_ref, thr_ref, h_ref):
        t = _reduce_norm(parts_ref[...], B)
        h_ref[...] = jax.nn.sigmoid(t - jnp.abs(thr_ref[...]))

    return pl.pallas_call(
        body,
        out_shape=jax.ShapeDtypeStruct((B, N), jnp.float32),
    )


def _make_final(B, N, NW, n_classes):
    def body(parts_ref, thr_ref, mask_ref, wfc_ref, bfc_ref, out_ref):
        t = _reduce_norm(parts_ref[...], B)
        h = jax.nn.sigmoid(t - jnp.abs(thr_ref[...]))
        pooled = jnp.sum(h * mask_ref[...], axis=1, keepdims=True)
        out_ref[...] = pooled * wfc_ref[...] + bfc_ref[...]

    return pl.pallas_call(
        body,
        out_shape=jax.ShapeDtypeStruct((B, n_classes), jnp.float32),
    )


def kernel(x, edge_index, edge_weight, edge_weight_multiplier,
           neuron_activation_threshold, W_fc, b_fc, sel_idx):
    N = neuron_activation_threshold.shape[0]
    B = x.shape[0] // N
    E = edge_weight.shape[0]
    S = sel_idx.shape[0]
    n_classes = W_fc.shape[0]
    NW = _NC * _NS

    h = x.reshape(B, N)
    src = edge_index[0]
    dst = edge_index[1]
    thr2 = neuron_activation_threshold.reshape(1, N)
    maskw = jnp.zeros((N,), jnp.float32).at[sel_idx].set(1.0 / S).reshape(1, N)
    wfc_row = W_fc.reshape(1, n_classes)
    bfc_row = b_fc.reshape(1, n_classes)

    C = _pick_chunk(E // (NW // B), N)
    pairw = _prep_edges(src, dst, edge_weight, edge_weight_multiplier, C)
    prop = _make_propagate(B, N, E)
    update = _make_update(B, N, NW)
    final = _make_final(B, N, NW, n_classes)

    for p in range(_NUM_PASSES):
        parts = prop(h, pairw)
        if p < _NUM_PASSES - 1:
            h = update(parts, thr2)
        else:
            out = final(parts, thr2, maskw, wfc_row, bfc_row)
    return out
